# speedup vs baseline: 2.4007x; 2.4007x over previous
import functools

import jax
import jax.numpy as jnp
from jax import lax
from jax.experimental import pallas as pl
from jax.experimental.pallas import tpu as pltpu
from jax.experimental.pallas import tpu_sc as plsc

B, S, D = 4, 8192, 768
E = 64
K = 8
SEU = B * S
L = 16

_S8 = [(0, 1), (2, 3), (4, 5), (6, 7),
       (0, 2), (1, 3), (4, 6), (5, 7),
       (1, 2), (5, 6),
       (0, 4), (1, 5), (2, 6), (3, 7),
       (2, 4), (3, 5),
       (1, 2), (3, 4), (5, 6)]


def _sort8_desc(v):
    v = list(v)
    for i, j in _S8:
        a, b = v[i], v[j]
        v[i] = jnp.maximum(a, b)
        v[j] = jnp.minimum(a, b)
    return v


def _merge_top8(a, b):
    t = [jnp.maximum(a[i], b[7 - i]) for i in range(8)]
    for d in (4, 2, 1):
        for i in range(8):
            if (i % (2 * d)) < d:
                hi = jnp.maximum(t[i], t[i + d])
                lo = jnp.minimum(t[i], t[i + d])
                t[i], t[i + d] = hi, lo
    return t


_TC_BLK = 1024


def _matmul_body(h_ref, w_ref, out_ref):
    out_ref[0] = lax.dot_general(
        w_ref[...], h_ref[...],
        dimension_numbers=(((1,), (1,)), ((), ())),
        preferred_element_type=jnp.float32)


def _logits_blocked(hflat, w, nw):
    grid = SEU // _TC_BLK
    return pl.pallas_call(
        _matmul_body,
        grid=(grid,),
        in_specs=[
            pl.BlockSpec((_TC_BLK, D), lambda i: (i, 0)),
            pl.BlockSpec((E, D), lambda i: (0, 0)),
        ],
        out_specs=pl.BlockSpec((1, E, _TC_BLK), lambda i: (i, 0, 0)),
        out_shape=jax.ShapeDtypeStruct((nw, E, SEU // nw), jnp.float32),
    )(hflat, w)


@functools.cache
def _make_sc_topk():
    info = plsc.get_sparse_core_info()
    nc, ns = info.num_cores, info.num_subcores
    nw = nc * ns
    rpw = SEU // nw
    groups = rpw // L
    blocks = E // K

    mesh = plsc.VectorSubcoreMesh(core_axis_name="c", subcore_axis_name="s")

    @functools.partial(
        pl.kernel,
        out_type=jax.ShapeDtypeStruct((nw, K, rpw), jnp.float32),
        mesh=mesh,
        scratch_types=[
            pltpu.VMEM((E, rpw), jnp.float32),
            pltpu.VMEM((K, rpw), jnp.float32),
        ],
    )
    def sc_topk(logits_hbm, out_hbm, lblk, oblk):
        wid = lax.axis_index("s") * nc + lax.axis_index("c")
        pltpu.sync_copy(logits_hbm.at[wid], lblk)

        def group_body(g, carry):
            col = g * L
            acc = _sort8_desc([lblk[e, pl.ds(col, L)] for e in range(K)])
            for blk in range(1, blocks):
                cand = _sort8_desc(
                    [lblk[blk * K + t, pl.ds(col, L)] for t in range(K)])
                acc = _merge_top8(acc, cand)
            exps = [jnp.exp(a - acc[0]) for a in acc]
            ssum = exps[0]
            for j in range(1, K):
                ssum = ssum + exps[j]
            inv = jnp.float32(1.0) / ssum
            for j in range(K):
                oblk[j, pl.ds(col, L)] = exps[j] * inv
            return carry

        lax.fori_loop(0, groups, group_body, 0)
        pltpu.sync_copy(oblk, out_hbm.at[wid])

    return sc_topk, nw


def kernel(hidden_states, kernel):
    sc_topk, nw = _make_sc_topk()
    hflat = hidden_states.reshape(SEU, D)
    logits = _logits_blocked(hflat, kernel, nw)
    out_blk = sc_topk(logits)
    return out_blk.transpose(0, 2, 1).reshape(SEU, K)


# speedup vs baseline: 2.7731x; 1.1551x over previous
import functools

import jax
import jax.numpy as jnp
from jax import lax
from jax.experimental import pallas as pl
from jax.experimental.pallas import tpu as pltpu
from jax.experimental.pallas import tpu_sc as plsc

B, S, D = 4, 8192, 768
E = 64
K = 8
SEU = B * S
L = 16

_S8 = [(0, 1), (2, 3), (4, 5), (6, 7),
       (0, 2), (1, 3), (4, 6), (5, 7),
       (1, 2), (5, 6),
       (0, 4), (1, 5), (2, 6), (3, 7),
       (2, 4), (3, 5),
       (1, 2), (3, 4), (5, 6)]


def _sort8_desc(v):
    v = list(v)
    for i, j in _S8:
        a, b = v[i], v[j]
        v[i] = jnp.maximum(a, b)
        v[j] = jnp.minimum(a, b)
    return v


def _merge_top8(a, b):
    t = [jnp.maximum(a[i], b[7 - i]) for i in range(8)]
    for d in (4, 2, 1):
        for i in range(8):
            if (i % (2 * d)) < d:
                hi = jnp.maximum(t[i], t[i + d])
                lo = jnp.minimum(t[i], t[i + d])
                t[i], t[i + d] = hi, lo
    return t


_TC_BLK = 4096
_RPW = 1024


def _matmul_body(h_ref, w_ref, out_ref):
    res = lax.dot_general(
        w_ref[...], h_ref[...],
        dimension_numbers=(((1,), (1,)), ((), ())),
        preferred_element_type=jnp.float32)
    for c in range(_TC_BLK // _RPW):
        out_ref[c] = res[:, c * _RPW:(c + 1) * _RPW]


def _logits_blocked(hflat, w, nw):
    grid = SEU // _TC_BLK
    sub = _TC_BLK // _RPW
    return pl.pallas_call(
        _matmul_body,
        grid=(grid,),
        in_specs=[
            pl.BlockSpec((_TC_BLK, D), lambda i: (i, 0)),
            pl.BlockSpec((E, D), lambda i: (0, 0)),
        ],
        out_specs=pl.BlockSpec((sub, E, _RPW), lambda i: (i, 0, 0)),
        out_shape=jax.ShapeDtypeStruct((nw, E, SEU // nw), jnp.float32),
    )(hflat, w)


@functools.cache
def _make_sc_topk():
    info = plsc.get_sparse_core_info()
    nc, ns = info.num_cores, info.num_subcores
    nw = nc * ns
    rpw = SEU // nw
    groups = rpw // L
    blocks = E // K

    mesh = plsc.VectorSubcoreMesh(core_axis_name="c", subcore_axis_name="s")

    @functools.partial(
        pl.kernel,
        out_type=jax.ShapeDtypeStruct((nw, K, rpw), jnp.float32),
        mesh=mesh,
        scratch_types=[
            pltpu.VMEM((E, rpw), jnp.float32),
            pltpu.VMEM((K, rpw), jnp.float32),
        ],
    )
    def sc_topk(logits_hbm, out_hbm, lblk, oblk):
        wid = lax.axis_index("s") * nc + lax.axis_index("c")
        pltpu.sync_copy(logits_hbm.at[wid], lblk)

        def group_body(g, carry):
            col = g * L
            acc = _sort8_desc([lblk[e, pl.ds(col, L)] for e in range(K)])
            for blk in range(1, blocks):
                cand = _sort8_desc(
                    [lblk[blk * K + t, pl.ds(col, L)] for t in range(K)])
                acc = _merge_top8(acc, cand)
            exps = [jnp.exp(a - acc[0]) for a in acc]
            ssum = exps[0]
            for j in range(1, K):
                ssum = ssum + exps[j]
            inv = jnp.float32(1.0) / ssum
            for j in range(K):
                oblk[j, pl.ds(col, L)] = exps[j] * inv
            return carry

        lax.fori_loop(0, groups, group_body, 0)
        pltpu.sync_copy(oblk, out_hbm.at[wid])

    return sc_topk, nw


def kernel(hidden_states, kernel):
    sc_topk, nw = _make_sc_topk()
    hflat = hidden_states.reshape(SEU, D)
    logits = _logits_blocked(hflat, kernel, nw)
    out_blk = sc_topk(logits)
    return out_blk.transpose(0, 2, 1).reshape(SEU, K)
